# baseline (speedup 1.0000x reference)
import jax
import jax.numpy as jnp
from jax import lax
from jax.experimental import pallas as pl
from jax.experimental.pallas import tpu as pltpu
from jax.experimental.pallas import tpu_sc as plsc

B = 16384
RANK = 32
NMODE = 3
PACK = 4
NC = 2
NS = 16
NW = NC * NS
BPW = B // NW
L = 16
CHUNK = 256
NCHUNK = BPW // CHUNK
CGROUPS = CHUNK // L


def _cpd_body(idx0_h, idx1_h, idx2_h, e0_h, e1_h, e2_h, out_h,
              i0, i1, i2, g0, g1, g2, rows0, rows1, rows2, out_v,
              sem0, sem1, sem2):
    wid = lax.axis_index("s") * NC + lax.axis_index("c")
    base = wid * BPW

    pltpu.sync_copy(idx0_h.at[pl.ds(base, BPW)], i0)
    pltpu.sync_copy(idx1_h.at[pl.ds(base, BPW)], i1)
    pltpu.sync_copy(idx2_h.at[pl.ds(base, BPW)], i2)

    ii = [i0, i1, i2]
    gg = [g0, g1, g2]
    tables = [e0_h, e1_h, e2_h]
    rows = [rows0, rows1, rows2]
    sems = [sem0, sem1, sem2]

    for c in range(NCHUNK):
        cbase = c * CHUNK

        def packrow(g, carry):
            src = pl.ds(cbase + g * L, L)
            dst = pl.ds(g * L, L)
            for m in range(NMODE):
                gg[m][dst] = lax.shift_right_logical(ii[m][src], 2)
            return carry

        lax.fori_loop(0, CGROUPS, packrow, 0)

        copies = [
            pltpu.async_copy(tables[m].at[gg[m]], rows[m], sems[m])
            for m in range(NMODE)
        ]
        for cp in copies:
            cp.wait()

        def group(g, carry):
            row = g * L + lax.iota(jnp.int32, L)
            sl = pl.ds(cbase + g * L, L)
            col0 = lax.shift_left(jnp.bitwise_and(i0[sl], 3), 5)
            col1 = lax.shift_left(jnp.bitwise_and(i1[sl], 3), 5)
            col2 = lax.shift_left(jnp.bitwise_and(i2[sl], 3), 5)
            acc = jnp.zeros((L,), jnp.float32)
            for r in range(RANK):
                a = plsc.load_gather(rows0, [row, col0 + r])
                b = plsc.load_gather(rows1, [row, col1 + r])
                cc = plsc.load_gather(rows2, [row, col2 + r])
                acc = acc + a * b * cc
            out_v[sl] = acc
            return carry

        lax.fori_loop(0, CGROUPS, group, 0)

    pltpu.sync_copy(out_v, out_h.at[pl.ds(base, BPW)])


def kernel(idxs, E0, E1, E2):
    idxs32 = idxs.astype(jnp.int32)
    idx0 = idxs32[:, 0]
    idx1 = idxs32[:, 1]
    idx2 = idxs32[:, 2]
    e0 = E0.reshape(-1, 128)
    e1 = E1.reshape(-1, 128)
    e2 = E2.reshape(-1, 128)
    mesh = plsc.VectorSubcoreMesh(core_axis_name="c", subcore_axis_name="s")
    f = pl.kernel(
        _cpd_body,
        out_type=jax.ShapeDtypeStruct((B,), jnp.float32),
        mesh=mesh,
        compiler_params=pltpu.CompilerParams(needs_layout_passes=False),
        scratch_types=[
            pltpu.VMEM((BPW,), jnp.int32),
            pltpu.VMEM((BPW,), jnp.int32),
            pltpu.VMEM((BPW,), jnp.int32),
            pltpu.VMEM((CHUNK,), jnp.int32),
            pltpu.VMEM((CHUNK,), jnp.int32),
            pltpu.VMEM((CHUNK,), jnp.int32),
            pltpu.VMEM((CHUNK, 128), jnp.float32),
            pltpu.VMEM((CHUNK, 128), jnp.float32),
            pltpu.VMEM((CHUNK, 128), jnp.float32),
            pltpu.VMEM((BPW,), jnp.float32),
            pltpu.SemaphoreType.DMA,
            pltpu.SemaphoreType.DMA,
            pltpu.SemaphoreType.DMA,
        ],
    )
    return f(idx0, idx1, idx2, e0, e1, e2)


# speedup vs baseline: 1.0373x; 1.0373x over previous
import jax
import jax.numpy as jnp
from jax import lax
from jax.experimental import pallas as pl
from jax.experimental.pallas import tpu as pltpu
from jax.experimental.pallas import tpu_sc as plsc

B = 16384
RANK = 32
NMODE = 3
PACK = 4
NC = 2
NS = 16
NW = NC * NS
BPW = B // NW
L = 16
CHUNK = 128
NCHUNK = BPW // CHUNK
CGROUPS = CHUNK // L
NACC = 4


def _cpd_body(idx0_h, idx1_h, idx2_h, e0_h, e1_h, e2_h, out_h,
              i0, i1, i2,
              ga0, ga1, ga2, gb0, gb1, gb2,
              ra0, ra1, ra2, rb0, rb1, rb2,
              out_v,
              sa0, sa1, sa2, sb0, sb1, sb2):
    wid = lax.axis_index("s") * NC + lax.axis_index("c")
    base = wid * BPW

    pltpu.sync_copy(idx0_h.at[pl.ds(base, BPW)], i0)
    pltpu.sync_copy(idx1_h.at[pl.ds(base, BPW)], i1)
    pltpu.sync_copy(idx2_h.at[pl.ds(base, BPW)], i2)

    ii = [i0, i1, i2]
    tables = [e0_h, e1_h, e2_h]
    gsets = [[ga0, ga1, ga2], [gb0, gb1, gb2]]
    rsets = [[ra0, ra1, ra2], [rb0, rb1, rb2]]
    ssets = [[sa0, sa1, sa2], [sb0, sb1, sb2]]

    def issue(c):
        s = c & 1
        gg, rows, sems = gsets[s], rsets[s], ssets[s]
        cbase = c * CHUNK

        def packrow(g, carry):
            src = pl.ds(cbase + g * L, L)
            dst = pl.ds(g * L, L)
            for m in range(NMODE):
                gg[m][dst] = lax.shift_right_logical(ii[m][src], 2)
            return carry

        lax.fori_loop(0, CGROUPS, packrow, 0)
        return [
            pltpu.async_copy(tables[m].at[gg[m]], rows[m], sems[m])
            for m in range(NMODE)
        ]

    def compute(c):
        s = c & 1
        rows0, rows1, rows2 = rsets[s]
        cbase = c * CHUNK

        def group(g, carry):
            row = g * L + lax.iota(jnp.int32, L)
            sl = pl.ds(cbase + g * L, L)
            col0 = lax.shift_left(jnp.bitwise_and(i0[sl], 3), 5)
            col1 = lax.shift_left(jnp.bitwise_and(i1[sl], 3), 5)
            col2 = lax.shift_left(jnp.bitwise_and(i2[sl], 3), 5)
            accs = [jnp.zeros((L,), jnp.float32) for _ in range(NACC)]
            for r in range(RANK):
                a = plsc.load_gather(rows0, [row, col0 + r])
                b = plsc.load_gather(rows1, [row, col1 + r])
                cc = plsc.load_gather(rows2, [row, col2 + r])
                accs[r % NACC] = accs[r % NACC] + a * b * cc
            out_v[sl] = (accs[0] + accs[1]) + (accs[2] + accs[3])
            return carry

        lax.fori_loop(0, CGROUPS, group, 0)

    inflight = issue(0)
    for c in range(NCHUNK):
        for cp in inflight:
            cp.wait()
        if c + 1 < NCHUNK:
            nxt = issue(c + 1)
        else:
            nxt = []
        compute(c)
        inflight = nxt

    pltpu.sync_copy(out_v, out_h.at[pl.ds(base, BPW)])


def kernel(idxs, E0, E1, E2):
    idxs32 = idxs.astype(jnp.int32)
    idx0 = idxs32[:, 0]
    idx1 = idxs32[:, 1]
    idx2 = idxs32[:, 2]
    e0 = E0.reshape(-1, 128)
    e1 = E1.reshape(-1, 128)
    e2 = E2.reshape(-1, 128)
    mesh = plsc.VectorSubcoreMesh(core_axis_name="c", subcore_axis_name="s")
    f = pl.kernel(
        _cpd_body,
        out_type=jax.ShapeDtypeStruct((B,), jnp.float32),
        mesh=mesh,
        compiler_params=pltpu.CompilerParams(needs_layout_passes=False),
        scratch_types=(
            [pltpu.VMEM((BPW,), jnp.int32)] * 3
            + [pltpu.VMEM((CHUNK,), jnp.int32)] * 6
            + [pltpu.VMEM((CHUNK, 128), jnp.float32)] * 6
            + [pltpu.VMEM((BPW,), jnp.float32)]
            + [pltpu.SemaphoreType.DMA] * 6
        ),
    )
    return f(idx0, idx1, idx2, e0, e1, e2)
